# baseline (device time: 204219 ns/iter reference)
import jax
import jax.numpy as jnp
from jax import lax
from jax.experimental import pallas as pl
from jax.experimental.pallas import tpu as pltpu

NCHUNK = 8


def kernel(x):
    _, m, n2 = x.shape
    half = n2 // 2
    rows = m // NCHUNK

    def body(
        x_ref, out_ref, recv_ref, send_ref, stage_sems, keep_sems, send_sems, recv_sems
    ):
        my_x = lax.axis_index("x")
        my_y = lax.axis_index("y")
        my_z = lax.axis_index("z")
        partner = 1 - my_x

        barrier_sem = pltpu.get_barrier_semaphore()
        pl.semaphore_signal(
            barrier_sem,
            inc=1,
            device_id=(partner, my_y, my_z),
            device_id_type=pl.DeviceIdType.MESH,
        )
        pl.semaphore_wait(barrier_sem, 1)

        def exchange(keep_start, send_start):
            stages = []
            for i in range(NCHUNK):
                r = pl.ds(i * rows, rows)
                cp = pltpu.make_async_copy(
                    x_ref.at[0, r, pl.ds(send_start, half)],
                    send_ref.at[r, :],
                    stage_sems.at[i],
                )
                cp.start()
                stages.append(cp)
            keeps = []
            for i in range(NCHUNK):
                r = pl.ds(i * rows, rows)
                cp = pltpu.make_async_copy(
                    x_ref.at[0, r, pl.ds(keep_start, half)],
                    out_ref.at[r, :],
                    keep_sems.at[i],
                )
                cp.start()
                keeps.append(cp)
            rdmas = []
            for i in range(NCHUNK):
                r = pl.ds(i * rows, rows)
                stages[i].wait()
                rdma = pltpu.make_async_remote_copy(
                    src_ref=send_ref.at[r, :],
                    dst_ref=recv_ref.at[r, :],
                    send_sem=send_sems.at[i],
                    recv_sem=recv_sems.at[i],
                    device_id=(partner, my_y, my_z),
                    device_id_type=pl.DeviceIdType.MESH,
                )
                rdma.start()
                rdmas.append(rdma)
            for i in range(NCHUNK):
                keeps[i].wait()
            for i in range(NCHUNK):
                r = pl.ds(i * rows, rows)
                rdmas[i].wait_recv()
                out_ref[r, :] = out_ref[r, :] + recv_ref[r, :]
            for i in range(NCHUNK):
                rdmas[i].wait_send()

        @pl.when(my_x == 0)
        def _():
            exchange(0, half)

        @pl.when(my_x == 1)
        def _():
            exchange(half, 0)

    return pl.pallas_call(
        body,
        out_shape=jax.ShapeDtypeStruct((m, half), jnp.float32),
        in_specs=[pl.BlockSpec(memory_space=pl.ANY)],
        out_specs=pl.BlockSpec(memory_space=pltpu.VMEM),
        scratch_shapes=[
            pltpu.VMEM((m, half), jnp.float32),
            pltpu.VMEM((m, half), jnp.float32),
            pltpu.SemaphoreType.DMA((NCHUNK,)),
            pltpu.SemaphoreType.DMA((NCHUNK,)),
            pltpu.SemaphoreType.DMA((NCHUNK,)),
            pltpu.SemaphoreType.DMA((NCHUNK,)),
        ],
        compiler_params=pltpu.CompilerParams(
            collective_id=0,
            vmem_limit_bytes=56 * 1024 * 1024,
        ),
    )(x)
